# baseline (speedup 1.0000x reference)
import jax
import jax.numpy as jnp
from jax import lax
from jax.experimental import pallas as pl
from jax.experimental.pallas import tpu as pltpu
from jax.experimental.pallas import tpu_sc as plsc

NUM_EMB = 1000000
D = 32
SCALE_CONST = 10.0
BATCH = 16384
HIST = 50
B_TOTAL = BATCH * HIST

NC, NS, L = 2, 16, 16
NW = NC * NS

EBLK = 512
NBLK = (NUM_EMB + EBLK - 1) // EBLK

BPW = B_TOTAL // NW
CHUNK = 1024
NCHUNK = BPW // CHUNK


def _tc_transpose_body(wt_ref, out_ref):
    w3 = wt_ref[...].reshape(D, 4, 128)
    cols = [jnp.transpose(w3[:, q, :]) for q in range(4)]
    out_ref[...] = jnp.concatenate(cols, axis=1) * SCALE_CONST


def _gather_body(xf_hbm, table_hbm, out_hbm, idx_v, rows_v, sem):
    wid = lax.axis_index("s") * NC + lax.axis_index("c")
    base = wid * BPW

    @pl.loop(0, NCHUNK)
    def _chunk(g):
        off = base + g * CHUNK
        pltpu.sync_copy(xf_hbm.at[pl.ds(off, CHUNK)], idx_v)

        @pl.loop(0, CHUNK // L, unroll=8)
        def _perm(j):
            e = idx_v[pl.ds(j * L, L)]
            idx_v[pl.ds(j * L, L)] = (
                (e & -512) | ((e & 127) << 2) | ((e >> 7) & 3))

        pltpu.async_copy(table_hbm.at[idx_v], rows_v, sem).wait()
        pltpu.sync_copy(rows_v, out_hbm.at[pl.ds(off, CHUNK)])


def kernel(x, weight):
    wT = weight.T
    xf = x.astype(jnp.int32).T.reshape(B_TOTAL)

    table128 = pl.pallas_call(
        _tc_transpose_body,
        grid=(NBLK,),
        in_specs=[pl.BlockSpec((D, EBLK), lambda k: (0, k))],
        out_specs=pl.BlockSpec((EBLK // 4, 128), lambda k: (k, 0)),
        out_shape=jax.ShapeDtypeStruct((NBLK * EBLK // 4, 128), jnp.float32),
    )(wT)

    table = table128.reshape(NBLK * EBLK, D)

    mesh = plsc.VectorSubcoreMesh(core_axis_name="c", subcore_axis_name="s")
    out2 = pl.kernel(
        _gather_body,
        out_type=jax.ShapeDtypeStruct((B_TOTAL, D), jnp.float32),
        mesh=mesh,
        scratch_types=[
            pltpu.VMEM((CHUNK,), jnp.int32),
            pltpu.VMEM((CHUNK, D), jnp.float32),
            pltpu.SemaphoreType.DMA,
        ],
        compiler_params=pltpu.CompilerParams(use_tc_tiling_on_sc=False),
    )(xf, table)

    return out2.reshape(HIST, BATCH, D).transpose(1, 0, 2)


# speedup vs baseline: 2.0921x; 2.0921x over previous
import jax
import jax.numpy as jnp
from jax import lax
from jax.experimental import pallas as pl
from jax.experimental.pallas import tpu as pltpu
from jax.experimental.pallas import tpu_sc as plsc

NUM_EMB = 1000000
D = 32
SCALE_CONST = 10.0
BATCH = 16384
HIST = 50
B_TOTAL = BATCH * HIST

NC, NS, L = 2, 16, 16
NW = NC * NS

EBLK = 512
GRP = 8
SBLK = EBLK * GRP
NSTEP = (NUM_EMB + SBLK - 1) // SBLK
NBLK = NSTEP * GRP

BPW = B_TOTAL // NW
CHUNK = 1024
NCHUNK = BPW // CHUNK


def _tc_transpose_body(wt_ref, out_ref):
    w3 = wt_ref[...].reshape(D, GRP * 4, 128)
    pieces = []
    for sb in range(GRP):
        cols = [jnp.transpose(w3[:, sb * 4 + q, :]) for q in range(4)]
        pieces.append(jnp.concatenate(cols, axis=1))
    out_ref[...] = jnp.concatenate(pieces, axis=0) * SCALE_CONST


def _gather_body(xf_hbm, table_hbm, out_hbm, idx_v, rows_v, sem):
    wid = lax.axis_index("s") * NC + lax.axis_index("c")
    base = wid * BPW

    @pl.loop(0, NCHUNK)
    def _chunk(g):
        off = base + g * CHUNK
        pltpu.sync_copy(xf_hbm.at[pl.ds(off, CHUNK)], idx_v)

        @pl.loop(0, CHUNK // L, unroll=8)
        def _perm(j):
            e = idx_v[pl.ds(j * L, L)]
            idx_v[pl.ds(j * L, L)] = (
                (e & -512) | ((e & 127) << 2) | ((e >> 7) & 3))

        pltpu.async_copy(table_hbm.at[idx_v], rows_v, sem).wait()
        pltpu.sync_copy(rows_v, out_hbm.at[pl.ds(off, CHUNK)])


def kernel(x, weight):
    wT = weight.T
    xf = x.astype(jnp.int32).T.reshape(B_TOTAL)

    table128 = pl.pallas_call(
        _tc_transpose_body,
        grid=(NSTEP,),
        in_specs=[pl.BlockSpec((D, SBLK), lambda k: (0, k))],
        out_specs=pl.BlockSpec((SBLK // 4, 128), lambda k: (k, 0)),
        out_shape=jax.ShapeDtypeStruct((NBLK * EBLK // 4, 128), jnp.float32),
    )(wT)

    table = table128.reshape(NBLK * EBLK, D)

    mesh = plsc.VectorSubcoreMesh(core_axis_name="c", subcore_axis_name="s")
    out2 = pl.kernel(
        _gather_body,
        out_type=jax.ShapeDtypeStruct((B_TOTAL, D), jnp.float32),
        mesh=mesh,
        scratch_types=[
            pltpu.VMEM((CHUNK,), jnp.int32),
            pltpu.VMEM((CHUNK, D), jnp.float32),
            pltpu.SemaphoreType.DMA,
        ],
        compiler_params=pltpu.CompilerParams(use_tc_tiling_on_sc=False),
    )(xf, table)

    return out2.reshape(HIST, BATCH, D).transpose(1, 0, 2)


# speedup vs baseline: 2.2176x; 1.0600x over previous
import jax
import jax.numpy as jnp
from jax import lax
from jax.experimental import pallas as pl
from jax.experimental.pallas import tpu as pltpu
from jax.experimental.pallas import tpu_sc as plsc

NUM_EMB = 1000000
D = 32
SCALE_CONST = 10.0
BATCH = 16384
HIST = 50
B_TOTAL = BATCH * HIST

NC, NS, L = 2, 16, 16
NW = NC * NS

EBLK = 512
GRP = 16
SBLK = EBLK * GRP
NSTEP = (NUM_EMB + SBLK - 1) // SBLK
NBLK = NSTEP * GRP

BPW = B_TOTAL // NW
CHUNK = 1024
NCHUNK = BPW // CHUNK


def _tc_transpose_body(wt_ref, out_ref):
    w3 = wt_ref[...].reshape(D, GRP * 4, 128)
    pieces = []
    for sb in range(GRP):
        cols = [jnp.transpose(w3[:, sb * 4 + q, :]) for q in range(4)]
        pieces.append(jnp.concatenate(cols, axis=1))
    out_ref[...] = jnp.concatenate(pieces, axis=0) * SCALE_CONST


def _gather_body(xf_hbm, table_hbm, out_hbm, idx_v, rows_v, sem):
    wid = lax.axis_index("s") * NC + lax.axis_index("c")
    base = wid * BPW

    @pl.loop(0, NCHUNK)
    def _chunk(g):
        off = base + g * CHUNK
        pltpu.sync_copy(xf_hbm.at[pl.ds(off, CHUNK)], idx_v)

        @pl.loop(0, CHUNK // L, unroll=8)
        def _perm(j):
            e = idx_v[pl.ds(j * L, L)]
            idx_v[pl.ds(j * L, L)] = (
                (e & -512) | ((e & 127) << 2) | ((e >> 7) & 3))

        pltpu.async_copy(table_hbm.at[idx_v], rows_v, sem).wait()
        pltpu.sync_copy(rows_v, out_hbm.at[pl.ds(off, CHUNK)])


def kernel(x, weight):
    wT = weight.T
    xf = x.astype(jnp.int32).T.reshape(B_TOTAL)

    table128 = pl.pallas_call(
        _tc_transpose_body,
        grid=(NSTEP,),
        in_specs=[pl.BlockSpec((D, SBLK), lambda k: (0, k))],
        out_specs=pl.BlockSpec((SBLK // 4, 128), lambda k: (k, 0)),
        out_shape=jax.ShapeDtypeStruct((NBLK * EBLK // 4, 128), jnp.float32),
    )(wT)

    table = table128.reshape(NBLK * EBLK, D)

    mesh = plsc.VectorSubcoreMesh(core_axis_name="c", subcore_axis_name="s")
    out2 = pl.kernel(
        _gather_body,
        out_type=jax.ShapeDtypeStruct((B_TOTAL, D), jnp.float32),
        mesh=mesh,
        scratch_types=[
            pltpu.VMEM((CHUNK,), jnp.int32),
            pltpu.VMEM((CHUNK, D), jnp.float32),
            pltpu.SemaphoreType.DMA,
        ],
        compiler_params=pltpu.CompilerParams(use_tc_tiling_on_sc=False),
    )(xf, table)

    return out2.reshape(HIST, BATCH, D).transpose(1, 0, 2)


# speedup vs baseline: 2.2335x; 1.0071x over previous
import jax
import jax.numpy as jnp
from jax import lax
from jax.experimental import pallas as pl
from jax.experimental.pallas import tpu as pltpu
from jax.experimental.pallas import tpu_sc as plsc

NUM_EMB = 1000000
D = 32
SCALE_CONST = 10.0
BATCH = 16384
HIST = 50
B_TOTAL = BATCH * HIST

NC, NS, L = 2, 16, 16
NW = NC * NS

EBLK = 512
GRP = 32
SBLK = EBLK * GRP
NSTEP = (NUM_EMB + SBLK - 1) // SBLK
NBLK = NSTEP * GRP

BPW = B_TOTAL // NW
CHUNK = 1024
NCHUNK = BPW // CHUNK


def _tc_transpose_body(wt_ref, out_ref):
    w3 = wt_ref[...].reshape(D, GRP * 4, 128)
    pieces = []
    for sb in range(GRP):
        cols = [jnp.transpose(w3[:, sb * 4 + q, :]) for q in range(4)]
        pieces.append(jnp.concatenate(cols, axis=1))
    out_ref[...] = jnp.concatenate(pieces, axis=0) * SCALE_CONST


def _gather_body(xf_hbm, table_hbm, out_hbm, idx_v, rows_v, sem):
    wid = lax.axis_index("s") * NC + lax.axis_index("c")
    base = wid * BPW

    @pl.loop(0, NCHUNK)
    def _chunk(g):
        off = base + g * CHUNK
        pltpu.sync_copy(xf_hbm.at[pl.ds(off, CHUNK)], idx_v)

        @pl.loop(0, CHUNK // L, unroll=8)
        def _perm(j):
            e = idx_v[pl.ds(j * L, L)]
            idx_v[pl.ds(j * L, L)] = (
                (e & -512) | ((e & 127) << 2) | ((e >> 7) & 3))

        pltpu.async_copy(table_hbm.at[idx_v], rows_v, sem).wait()
        pltpu.sync_copy(rows_v, out_hbm.at[pl.ds(off, CHUNK)])


def kernel(x, weight):
    wT = weight.T
    xf = x.astype(jnp.int32).T.reshape(B_TOTAL)

    table128 = pl.pallas_call(
        _tc_transpose_body,
        grid=(NSTEP,),
        in_specs=[pl.BlockSpec((D, SBLK), lambda k: (0, k))],
        out_specs=pl.BlockSpec((SBLK // 4, 128), lambda k: (k, 0)),
        out_shape=jax.ShapeDtypeStruct((NBLK * EBLK // 4, 128), jnp.float32),
    )(wT)

    table = table128.reshape(NBLK * EBLK, D)

    mesh = plsc.VectorSubcoreMesh(core_axis_name="c", subcore_axis_name="s")
    out2 = pl.kernel(
        _gather_body,
        out_type=jax.ShapeDtypeStruct((B_TOTAL, D), jnp.float32),
        mesh=mesh,
        scratch_types=[
            pltpu.VMEM((CHUNK,), jnp.int32),
            pltpu.VMEM((CHUNK, D), jnp.float32),
            pltpu.SemaphoreType.DMA,
        ],
        compiler_params=pltpu.CompilerParams(use_tc_tiling_on_sc=False),
    )(xf, table)

    return out2.reshape(HIST, BATCH, D).transpose(1, 0, 2)
